# baseline (speedup 1.0000x reference)
import functools

import jax
import jax.numpy as jnp
from jax import lax
from jax.experimental import pallas as pl
from jax.experimental.pallas import tpu as pltpu
from jax.experimental.pallas import tpu_sc as plsc

_N_RAYS = 100000

_NC, _NS = 2, 16
_NW = _NC * _NS

_WC = 125
_CH = 16 * _WC
_WINS = 100
_TILE_SAMP = _WINS * _CH
_UNROLL = 5

_ACC = 102656
_DUMMY = _ACC - 1
_SROWS = 896


def _sc_partial_sums(idx1d, w1d):
    assert idx1d.shape[0] == _NW * _TILE_SAMP
    assert _WC % _UNROLL == 0

    mesh = plsc.VectorSubcoreMesh(core_axis_name="c", subcore_axis_name="s")

    @functools.partial(
        pl.kernel,
        out_type=jax.ShapeDtypeStruct((_NC, _SROWS, 128), jnp.float32),
        mesh=mesh,
        compiler_params=pltpu.CompilerParams(needs_layout_passes=False),
        scratch_types=[
            pltpu.VMEM((3 * _CH,), jnp.int32),
            pltpu.VMEM((3 * _CH,), jnp.float32),
            pltpu.VMEM((_ACC // 128, 128), jnp.float32),
            pltpu.VMEM((16,), jnp.int32),
            pltpu.VMEM_SHARED((_SROWS, 128), jnp.float32),
            pltpu.SemaphoreType.DMA((3,)),
        ],
    )
    def k(idx_hbm, w_hbm, out_hbm, ibuf, wbuf, acc, tailbuf, spacc, sem):
        c = lax.axis_index("c")
        s = lax.axis_index("s")
        tid = c * _NS + s
        tbase = tid * _TILE_SAMP

        lane_id = lax.iota(jnp.int32, 16)
        zeros16 = jnp.zeros((16,), jnp.float32)
        dummy16 = jnp.full((16,), _DUMMY, jnp.int32)

        def issue(w, p):
            base = tbase + w * _CH
            pltpu.make_async_copy(idx_hbm.at[pl.ds(base, _CH)],
                                  ibuf.at[pl.ds(p * _CH, _CH)],
                                  sem.at[p]).start()
            pltpu.make_async_copy(w_hbm.at[pl.ds(base, _CH)],
                                  wbuf.at[pl.ds(p * _CH, _CH)],
                                  sem.at[p]).start()

        def wait_win(w, p):
            base = tbase + w * _CH
            pltpu.make_async_copy(idx_hbm.at[pl.ds(base, _CH)],
                                  ibuf.at[pl.ds(p * _CH, _CH)],
                                  sem.at[p]).wait()
            pltpu.make_async_copy(w_hbm.at[pl.ds(base, _CH)],
                                  wbuf.at[pl.ds(p * _CH, _CH)],
                                  sem.at[p]).wait()

        def run_window(p, wc):
            lane_off = lane_id * wc + p * _CH

            def step_block(i, carry):
                cur, accv, base = carry
                t = i * _UNROLL
                loaded = []
                for u in range(_UNROLL):
                    ivec = lane_off + (t + u)
                    loaded.append((plsc.load_gather(ibuf, [ivec]),
                                   plsc.load_gather(wbuf, [ivec])))
                for idxv, wvv in loaded:
                    m = idxv != cur
                    plsc.addupdate_scatter(
                        acc,
                        [lax.shift_right_logical(cur, 7),
                         lax.bitwise_and(cur, 127)],
                        accv - base, mask=m)
                    base = jnp.where(m, accv, base)
                    accv = accv + wvv
                    cur = idxv
                return cur, accv, base

            cur, accv, base = lax.fori_loop(0, wc // _UNROLL, step_block,
                                            (dummy16, zeros16, zeros16))
            row = lax.shift_right_logical(cur, 7)
            col = lax.bitwise_and(cur, 127)
            for i in range(16):
                plsc.addupdate_scatter(acc, [row, col], accv - base,
                                       mask=lane_id == i)

        issue(0, 0)
        issue(1, 1)
        pltpu.sync_copy(idx_hbm.at[pl.ds(tbase + _TILE_SAMP - 16, 16)],
                        tailbuf)
        wait_win(0, 0)
        lo = ibuf[pl.ds(0, 16)][0]
        hi = tailbuf[...][15]
        r0 = lax.shift_right_logical(lo, 7)
        r1 = lax.shift_right_logical(hi, 7)
        cnt = lax.shift_right_logical(r1 - r0, 4) + 1

        def zchunk(k_, _):
            rb = r0 + k_ * 16
            for r in range(16):
                for u in range(8):
                    acc[rb + r, pl.ds(u * 16, 16)] = zeros16
            return 0

        lax.fori_loop(0, cnt, zchunk, 0)

        for k_ in range(4):
            pltpu.sync_copy(acc.at[pl.ds(r0, 14)],
                            spacc.at[pl.ds(s * 56 + k_ * 14, 14)])
        plsc.subcore_barrier()

        issue(2, 2)
        run_window(0, _WC)

        def wloop(w, _):
            p = w % 3
            wait_win(w, p)

            @pl.when(w + 2 < _WINS)
            def _():
                issue(w + 2, (w + 2) % 3)

            run_window(p, _WC)
            return 0

        lax.fori_loop(1, _WINS, wloop, 0)

        def mchunk(k_, _):
            rb = r0 + k_ * 16
            rowvec = lane_id + rb
            pltpu.sync_copy(acc.at[pl.ds(rb, 16)], spacc.at[rowvec], add=True)
            return 0

        lax.fori_loop(0, cnt, mchunk, 0)
        plsc.subcore_barrier()

        pltpu.sync_copy(spacc.at[pl.ds(s * 56, 56)],
                        out_hbm.at[c, pl.ds(s * 56, 56)])

    return k(idx1d, w1d)


def _combine_body(p_ref, o_ref):
    o_ref[...] = jnp.clip(jnp.sum(p_ref[...], axis=0), 0.0, 1.0)


def kernel(weights, ray_indices, num_rays):
    del num_rays
    idx = ray_indices.astype(jnp.int32)
    w = weights.astype(jnp.float32)

    partial = _sc_partial_sums(idx, w)

    out = pl.pallas_call(
        _combine_body,
        out_shape=jax.ShapeDtypeStruct((_SROWS, 128), jnp.float32),
    )(partial)
    return out.reshape(_SROWS * 128)[:_N_RAYS][:, None]
